# baseline (speedup 1.0000x reference)
import jax
import jax.numpy as jnp
from jax.experimental import pallas as pl

D = 128
N0, N1, N2, N3, N4 = 10000, 15000, 10000, 4000, 1000
NEG = 0.2


def _leaky(x):
    return jnp.where(x >= 0, x, NEG * x)


def _row_softmax(vals, rows, n_rows):
    m = jax.ops.segment_max(vals, rows, num_segments=n_rows)
    m = jnp.where(jnp.isfinite(m), m, 0.0)
    e = jnp.exp(vals - m[rows])
    s = jax.ops.segment_sum(e, rows, num_segments=n_rows)
    return e / jnp.maximum(s[rows], 1e-20)


def _hbs(x, idx, n, W, a):
    msg = x @ W
    i = idx[0]
    j = idx[1]
    z = jnp.concatenate([msg[i], msg[j]], axis=1)
    att = _row_softmax(_leaky(z @ a)[:, 0], i, n)
    return jax.ops.segment_sum(att[:, None] * msg[j], i, num_segments=n)


def _hbns(x_s, x_t, idx, n_t, n_s, Ws, Wt, a):
    s_msg = x_s @ Ws
    t_msg = x_t @ Wt
    ti = idx[0]
    sj = idx[1]
    e = _leaky(jnp.concatenate([s_msg[sj], t_msg[ti]], axis=1) @ a)[:, 0]
    f = _leaky(jnp.concatenate([t_msg[ti], s_msg[sj]], axis=1) @ a)[:, 0]
    e_att = _row_softmax(e, ti, n_t)
    f_att = _row_softmax(f, sj, n_s)
    msg_on_target = jax.ops.segment_sum(e_att[:, None] * s_msg[sj], ti, num_segments=n_t)
    msg_on_source = jax.ops.segment_sum(f_att[:, None] * t_msg[ti], sj, num_segments=n_s)
    return msg_on_source, msg_on_target


def _agg(lst):
    return jnp.mean(jnp.stack(lst, axis=0), axis=0)


def kernel(x_0, x_1, x_2, x_3, x_4, adjacency_0, adjacency_1, adjacency_2, adjacency_3, adjacency_4, incidence_1, incidence_2, incidence_3, incidence_4, W_hbs, A_hbs, Ws_hbns, Wt_hbns, A_hbns):
    x_1_to_1 = _hbs(x_1, adjacency_1, N1, W_hbs[0], A_hbs[0])
    x_2_to_2 = _hbs(x_2, adjacency_2, N2, W_hbs[1], A_hbs[1])
    x_3_to_3 = _hbs(x_3, adjacency_3, N3, W_hbs[2], A_hbs[2])
    x_4_to_4 = _hbs(x_4, adjacency_4, N4, W_hbs[3], A_hbs[3])
    x_0_to_1, _u = _hbns(x_1, x_0, incidence_1, N0, N1, Ws_hbns[0], Wt_hbns[0], A_hbns[0])
    x_1_to_2, _u = _hbns(x_2, x_1, incidence_2, N1, N2, Ws_hbns[1], Wt_hbns[1], A_hbns[1])
    x_2_to_3, _u = _hbns(x_3, x_2, incidence_3, N2, N3, Ws_hbns[2], Wt_hbns[2], A_hbns[2])
    x_3_to_4, _u = _hbns(x_4, x_3, incidence_4, N3, N4, Ws_hbns[3], Wt_hbns[3], A_hbns[3])
    x_1_level1 = _agg([x_0_to_1, x_1_to_1])
    x_2_level1 = _agg([x_1_to_2, x_2_to_2])
    x_3_level1 = _agg([x_2_to_3, x_3_to_3])
    x_4_level1 = _agg([x_3_to_4, x_4_to_4])
    x_2_to_2 = _hbs(x_2_level1, adjacency_2, N2, W_hbs[4], A_hbs[4])
    x_3_to_3 = _hbs(x_3_level1, adjacency_3, N3, W_hbs[5], A_hbs[5])
    x_1_to_2, _u = _hbns(x_2_level1, x_1_level1, incidence_2, N1, N2, Ws_hbns[4], Wt_hbns[4], A_hbns[4])
    x_2_to_3, _u = _hbns(x_3_level1, x_2_level1, incidence_3, N2, N3, Ws_hbns[5], Wt_hbns[5], A_hbns[5])
    _u, x_4_to_3 = _hbns(x_4_level1, x_3_level1, incidence_4, N3, N4, Ws_hbns[6], Wt_hbns[6], A_hbns[6])
    x_2_level2 = _agg([x_1_to_2, x_2_to_2])
    x_3_level2 = _agg([x_2_to_3, x_3_to_3, x_4_to_3])
    x_4_level2 = x_4_level1
    return (x_0, x_1_level1, x_2_level2, x_3_level2, x_4_level2)


# speedup vs baseline: 5.8942x; 5.8942x over previous
import functools

import jax
import jax.numpy as jnp
from jax import lax
from jax.experimental import pallas as pl
from jax.experimental.pallas import tpu as pltpu
from jax.experimental.pallas import tpu_sc as plsc

F32 = jnp.float32
D = 128
NEG = 0.2
TILES = 32
EPB = 128


def _pad16(n):
    return (n // 128 + 1) * 128


@functools.cache
def _make_edge_kernel(ng, acc, nblocks, rowsplit):
    chunk = acc // 16
    mesh = plsc.VectorSubcoreMesh(core_axis_name="c", subcore_axis_name="s",
                                  num_cores=2, num_subcores=16)

    def body(v_tab, u_h, v_h, r_h, g_h, num_o, den_o,
             u_v, vv_v, den_v, rblk, gblk, e_v, rows_v, num_sh):
        c = lax.axis_index("c")
        s = lax.axis_index("s")
        wid = s * 2 + c
        cid = s if rowsplit else wid
        off = c * acc if rowsplit else 0

        pltpu.sync_copy(u_h.at[c], u_v)
        pltpu.sync_copy(v_h, vv_v)

        zf = jnp.zeros((16,), F32)

        def zden(i, _):
            den_v[pl.ds(i * 16, 16)] = zf
            return 0
        lax.fori_loop(0, acc // 16, zden, 0)

        def zrow(i, _):
            for cc in range(8):
                rows_v[i, pl.ds(cc * 16, 16)] = zf
            return 0
        lax.fori_loop(0, EPB, zrow, 0)

        base = s * chunk
        zoff = 0
        while zoff < chunk:
            sz = min(EPB, chunk - zoff)
            pltpu.sync_copy(rows_v.at[pl.ds(0, sz)],
                            num_sh.at[pl.ds(base + zoff, sz)])
            zoff += sz
        plsc.subcore_barrier()

        def blk(j, _):
            pltpu.sync_copy(r_h.at[cid, j], rblk)
            pltpu.sync_copy(g_h.at[cid, j], gblk)
            pltpu.sync_copy(v_tab.at[gblk], rows_v)
            for grp in range(8):
                sl = pl.ds(grp * 16, 16)
                r16 = rblk[sl]
                g16 = gblk[sl]
                loc = r16 - off
                ok = (loc >= 0) & (loc < acc)
                lidx = jnp.where(ok, loc, acc - 1)
                uu = plsc.load_gather(u_v, [lidx])
                vv = plsc.load_gather(vv_v, [g16])
                l = uu + vv
                e = jnp.exp(jnp.where(l >= 0, l, NEG * l))
                e = jnp.where(ok, e, 0.0)
                e_v[sl] = e
                plsc.addupdate_scatter(den_v, [lidx], e)
                rblk[sl] = lidx

            def scale(k, _):
                eb = plsc.load_gather(e_v, [jnp.full((16,), 0, jnp.int32) + k])
                for cc in range(8):
                    sl = pl.ds(cc * 16, 16)
                    rows_v[k, sl] = rows_v[k, sl] * eb
                return 0
            lax.fori_loop(0, EPB, scale, 0)

            pltpu.sync_copy(rows_v, num_sh.at[rblk], add=True)
            return 0
        lax.fori_loop(0, nblocks, blk, 0)

        plsc.subcore_barrier()
        pltpu.sync_copy(den_v, den_o.at[wid])
        zoff = 0
        while zoff < chunk:
            sz = min(512, chunk - zoff)
            pltpu.sync_copy(num_sh.at[pl.ds(base + zoff, sz)],
                            num_o.at[c, pl.ds(base + zoff, sz)])
            zoff += sz

    return pl.kernel(
        body,
        out_type=(jax.ShapeDtypeStruct((2, acc, D), F32),
                  jax.ShapeDtypeStruct((TILES, acc), F32)),
        mesh=mesh,
        compiler_params=pltpu.CompilerParams(needs_layout_passes=False),
        scratch_types=(
            pltpu.VMEM((acc,), F32),
            pltpu.VMEM((ng,), F32),
            pltpu.VMEM((acc,), F32),
            pltpu.VMEM((EPB,), jnp.int32),
            pltpu.VMEM((EPB,), jnp.int32),
            pltpu.VMEM((EPB,), F32),
            pltpu.VMEM((EPB, D), F32),
            pltpu.VMEM_SHARED((acc, D), F32),
        ),
    )


_ROWSPLIT_ABOVE = 8192


def _edge_op(v_tab, u, v, r_idx, g_idx, nr, token=None):
    ng = v_tab.shape[0]
    nnz = r_idx.shape[0]
    rowsplit = _pad16(nr) > _ROWSPLIT_ABOVE
    if rowsplit:
        acc = _pad16((nr + 1) // 2)
        chunks = 16
    else:
        acc = _pad16(nr)
        chunks = TILES
    per = chunks * EPB
    nnz_pad = ((nnz + per - 1) // per) * per
    nblocks = nnz_pad // per
    r_p = jnp.concatenate(
        [r_idx, jnp.full((nnz_pad - nnz,), nr, jnp.int32)]).reshape(
        chunks, nblocks, EPB)
    g_p = jnp.concatenate(
        [g_idx, jnp.zeros((nnz_pad - nnz,), jnp.int32)]).reshape(
        chunks, nblocks, EPB)
    u_p = jnp.pad(u, (0, 2 * acc - nr)) if rowsplit else jnp.pad(
        u, (0, acc - nr))
    u_p = (u_p.reshape(2, acc) if rowsplit
           else jnp.stack([u_p, u_p]))
    if token is not None:
        u_p, _ = lax.optimization_barrier((u_p, token))
    num, den = _make_edge_kernel(ng, acc, nblocks, rowsplit)(
        v_tab, u_p, v, r_p, g_p)
    if rowsplit:
        return (num.reshape(2 * acc, D),
                jnp.concatenate([den[0::2].sum(axis=0), den[1::2].sum(axis=0)]))
    return num[0] + num[1], den.sum(axis=0)


def _fold(W, a):
    return W @ a.reshape(2, D).T


def kernel(x_0, x_1, x_2, x_3, x_4, adjacency_0, adjacency_1, adjacency_2,
           adjacency_3, adjacency_4, incidence_1, incidence_2, incidence_3,
           incidence_4, W_hbs, A_hbs, Ws_hbns, Wt_hbns, A_hbns):
    n0, n1, n2, n3, n4 = (x_0.shape[0], x_1.shape[0], x_2.shape[0],
                          x_3.shape[0], x_4.shape[0])

    tok = [None]

    def run_edge(v_tab, u, v, r_idx, g_idx, nr):
        num, den = _edge_op(v_tab, u, v, r_idx, g_idx, nr, tok[0])
        tok[0] = den[0]
        return num[:nr] / jnp.maximum(den[:nr], 1e-20)[:, None]

    def hbs(x, adj, n, k):
        msg = x @ W_hbs[k]
        ca = _fold(W_hbs[k], A_hbs[k])
        u = x @ ca[:, 0]
        v = x @ ca[:, 1]
        return run_edge(msg, u, v, adj[0], adj[1], n)

    def hbns_src(x_s, x_t, inc, n_t_sz, n_s_sz, k):
        t_msg = x_t @ Wt_hbns[k]
        cs = _fold(Ws_hbns[k], A_hbns[k])
        ct = _fold(Wt_hbns[k], A_hbns[k])
        u = x_s @ cs[:, 1]
        v = x_t @ ct[:, 0]
        return run_edge(t_msg, u, v, inc[1], inc[0], n_s_sz)

    def hbns_tgt(x_s, x_t, inc, n_t_sz, n_s_sz, k):
        s_msg = x_s @ Ws_hbns[k]
        cs = _fold(Ws_hbns[k], A_hbns[k])
        ct = _fold(Wt_hbns[k], A_hbns[k])
        u = x_t @ ct[:, 1]
        v = x_s @ cs[:, 0]
        return run_edge(s_msg, u, v, inc[0], inc[1], n_t_sz)

    x_1_to_1 = hbs(x_1, adjacency_1, n1, 0)
    x_2_to_2 = hbs(x_2, adjacency_2, n2, 1)
    x_3_to_3 = hbs(x_3, adjacency_3, n3, 2)
    x_4_to_4 = hbs(x_4, adjacency_4, n4, 3)
    x_0_to_1 = hbns_src(x_1, x_0, incidence_1, n0, n1, 0)
    x_1_to_2 = hbns_src(x_2, x_1, incidence_2, n1, n2, 1)
    x_2_to_3 = hbns_src(x_3, x_2, incidence_3, n2, n3, 2)
    x_3_to_4 = hbns_src(x_4, x_3, incidence_4, n3, n4, 3)
    x_1_level1 = 0.5 * (x_0_to_1 + x_1_to_1)
    x_2_level1 = 0.5 * (x_1_to_2 + x_2_to_2)
    x_3_level1 = 0.5 * (x_2_to_3 + x_3_to_3)
    x_4_level1 = 0.5 * (x_3_to_4 + x_4_to_4)

    x_2_to_2b = hbs(x_2_level1, adjacency_2, n2, 4)
    x_3_to_3b = hbs(x_3_level1, adjacency_3, n3, 5)
    x_1_to_2b = hbns_src(x_2_level1, x_1_level1, incidence_2, n1, n2, 4)
    x_2_to_3b = hbns_src(x_3_level1, x_2_level1, incidence_3, n2, n3, 5)
    x_4_to_3b = hbns_tgt(x_4_level1, x_3_level1, incidence_4, n3, n4, 6)
    x_2_level2 = 0.5 * (x_1_to_2b + x_2_to_2b)
    x_3_level2 = (x_2_to_3b + x_3_to_3b + x_4_to_3b) / 3.0
    x_4_level2 = x_4_level1

    return (x_0, x_1_level1, x_2_level2, x_3_level2, x_4_level2)
